# baseline (speedup 1.0000x reference)
import functools

import jax
import jax.numpy as jnp
from jax import lax
from jax.experimental import pallas as pl
from jax.experimental.pallas import tpu as pltpu
from jax.experimental.pallas import tpu_sc as plsc

B = 16384
F = 32
V = 1_000_000
BV = 512
NBLK = (V + BV - 1) // BV
VP = NBLK * (BV // 4)
NW = 32
BPW = B // NW
CHUNK = 128
NCHUNK = BPW // CHUNK
L = 16


def _tc_repack(tab_t, eye):
    def body(x_ref, eye_ref, o_ref):
        e = eye_ref[...]
        parts = []
        for a in range(4):
            c = x_ref[:, a * 128:(a + 1) * 128]
            parts.append(jax.lax.dot_general(
                e, c, (((1,), (1,)), ((), ())),
                preferred_element_type=jnp.float32))
        o_ref[...] = jnp.concatenate(parts, axis=1)

    return pl.pallas_call(
        body,
        grid=(NBLK,),
        in_specs=[
            pl.BlockSpec((32, BV), lambda j: (0, j)),
            pl.BlockSpec((128, 128), lambda j: (0, 0)),
        ],
        out_specs=pl.BlockSpec((128, 128), lambda j: (j, 0)),
        out_shape=jax.ShapeDtypeStruct((VP, 128), jnp.float32),
    )(tab_t, eye)


def _make_sc_kernel():
    mesh = plsc.VectorSubcoreMesh(core_axis_name="c", subcore_axis_name="s")

    @functools.partial(
        pl.kernel,
        mesh=mesh,
        out_type=jax.ShapeDtypeStruct((B,), jnp.float32),
        compiler_params=pltpu.CompilerParams(
            needs_layout_passes=False, use_tc_tiling_on_sc=True),
        scratch_types=[
            pltpu.VMEM((NCHUNK, CHUNK), jnp.int32),
            pltpu.VMEM((NCHUNK, CHUNK), jnp.int32),
            pltpu.VMEM((NCHUNK, CHUNK), jnp.int32),
            pltpu.VMEM((NCHUNK, CHUNK), jnp.int32),
            pltpu.VMEM((CHUNK, 128), jnp.float32),
            pltpu.VMEM((CHUNK, 128), jnp.float32),
            pltpu.VMEM((BPW,), jnp.float32),
            pltpu.VMEM((3 * L,), jnp.float32),
            pltpu.SemaphoreType.DMA,
        ],
    )
    def sc_kernel(user_hbm, item_hbm, upk_hbm, ipk_hbm, params_hbm,
                  out_hbm, uidx, iidx, urow, irow, ubuf, ibuf, outv, pv, sem):
        wid = lax.axis_index("s") * 2 + lax.axis_index("c")

        pltpu.sync_copy(user_hbm.at[pl.ds(wid * NCHUNK, NCHUNK)], uidx)
        pltpu.sync_copy(item_hbm.at[pl.ds(wid * NCHUNK, NCHUNK)], iidx)
        pltpu.sync_copy(params_hbm, pv)

        iota = lax.iota(jnp.int32, L)
        for c in range(NCHUNK):
            for g in range(CHUNK // L):
                sl = pl.ds(g * L, L)
                iu = uidx[c, sl]
                ii = iidx[c, sl]
                urow[c, sl] = ((iu >> 9) << 7) | (iu & 127)
                irow[c, sl] = ((ii >> 9) << 7) | (ii & 127)

        w0 = pv[pl.ds(0, L)]
        w1 = pv[pl.ds(L, L)]
        bv = pv[pl.ds(2 * L, L)]

        for c in range(NCHUNK):
            cu = pltpu.async_copy(upk_hbm.at[urow.at[c]], ubuf, sem)
            ci = pltpu.async_copy(ipk_hbm.at[irow.at[c]], ibuf, sem)
            cu.wait()
            ci.wait()
            for g in range(CHUNK // L):
                sl = pl.ds(g * L, L)
                rows = g * L + iota
                au = ((uidx[c, sl] >> 7) & 3) << 5
                ai = ((iidx[c, sl] >> 7) & 3) << 5
                mf = jnp.zeros((L,), jnp.float32)
                d2 = jnp.zeros((L,), jnp.float32)
                for f in range(F):
                    u = plsc.load_gather(ubuf, [rows, au + f])
                    i = plsc.load_gather(ibuf, [rows, ai + f])
                    mf = mf + u * i
                    d = u - i
                    d2 = d2 + d * d
                bits = lax.bitcast_convert_type(d2, jnp.int32)
                r = lax.bitcast_convert_type(
                    jnp.int32(0x5F3759DF) - (bits >> 1), jnp.float32)
                for _ in range(3):
                    r = r * (1.5 - 0.5 * d2 * r * r)
                p2 = d2 * r
                outv[pl.ds(c * CHUNK + g * L, L)] = w0 * p2 + w1 * mf + bv

        pltpu.sync_copy(outv, out_hbm.at[pl.ds(wid * BPW, BPW)])

    return sc_kernel


_SC_KERNEL = _make_sc_kernel()


def kernel(user, item, user_emb, item_emb, W, b):
    eye = jnp.eye(128, dtype=jnp.float32)
    upk = _tc_repack(user_emb.T, eye)
    ipk = _tc_repack(item_emb.T, eye)
    user2d = user.reshape(NW * NCHUNK, CHUNK)
    item2d = item.reshape(NW * NCHUNK, CHUNK)
    params = jnp.concatenate([
        jnp.full((L,), W[0, 0], jnp.float32),
        jnp.full((L,), W[0, 1], jnp.float32),
        jnp.full((L,), b[0], jnp.float32),
    ])
    return _SC_KERNEL(user2d, item2d, upk, ipk, params)


# speedup vs baseline: 4.4188x; 4.4188x over previous
import functools

import jax
import jax.numpy as jnp
from jax import lax
from jax.experimental import pallas as pl
from jax.experimental.pallas import tpu as pltpu
from jax.experimental.pallas import tpu_sc as plsc

B = 16384
F = 32
V = 1_000_000
BV = 4096
NBLK = (V + BV - 1) // BV
VP = NBLK * (BV // 4)
NW = 32
BPW = B // NW
CHUNK = 128
NCHUNK = BPW // CHUNK
L = 16


def _tc_repack(tab_t, eye):
    def body(x_ref, eye_ref, o_ref):
        e = eye_ref[...]
        for s8 in range(BV // 512):
            parts = []
            for a in range(4):
                c = x_ref[:, (s8 * 4 + a) * 128:(s8 * 4 + a + 1) * 128]
                parts.append(jax.lax.dot_general(
                    e, c, (((1,), (1,)), ((), ())),
                    preferred_element_type=jnp.float32))
            o_ref[s8 * 128:(s8 + 1) * 128, :] = jnp.concatenate(
                parts, axis=1)

    return pl.pallas_call(
        body,
        grid=(NBLK,),
        in_specs=[
            pl.BlockSpec((32, BV), lambda j: (0, j)),
            pl.BlockSpec((128, 128), lambda j: (0, 0)),
        ],
        out_specs=pl.BlockSpec((BV // 4, 128), lambda j: (j, 0)),
        out_shape=jax.ShapeDtypeStruct((VP, 128), jnp.float32),
    )(tab_t, eye)


def _make_sc_kernel():
    mesh = plsc.VectorSubcoreMesh(core_axis_name="c", subcore_axis_name="s")

    @functools.partial(
        pl.kernel,
        mesh=mesh,
        out_type=jax.ShapeDtypeStruct((B,), jnp.float32),
        compiler_params=pltpu.CompilerParams(
            needs_layout_passes=False, use_tc_tiling_on_sc=True),
        scratch_types=[
            pltpu.VMEM((NCHUNK, CHUNK), jnp.int32),
            pltpu.VMEM((NCHUNK, CHUNK), jnp.int32),
            pltpu.VMEM((NCHUNK, CHUNK), jnp.int32),
            pltpu.VMEM((NCHUNK, CHUNK), jnp.int32),
            pltpu.VMEM((CHUNK, 128), jnp.float32),
            pltpu.VMEM((CHUNK, 128), jnp.float32),
            pltpu.VMEM((BPW,), jnp.float32),
            pltpu.VMEM((3 * L,), jnp.float32),
            pltpu.SemaphoreType.DMA,
        ],
    )
    def sc_kernel(user_hbm, item_hbm, upk_hbm, ipk_hbm, params_hbm,
                  out_hbm, uidx, iidx, urow, irow, ubuf, ibuf, outv, pv, sem):
        wid = lax.axis_index("s") * 2 + lax.axis_index("c")

        pltpu.sync_copy(user_hbm.at[pl.ds(wid * NCHUNK, NCHUNK)], uidx)
        pltpu.sync_copy(item_hbm.at[pl.ds(wid * NCHUNK, NCHUNK)], iidx)
        pltpu.sync_copy(params_hbm, pv)

        iota = lax.iota(jnp.int32, L)
        for c in range(NCHUNK):
            for g in range(CHUNK // L):
                sl = pl.ds(g * L, L)
                iu = uidx[c, sl]
                ii = iidx[c, sl]
                urow[c, sl] = ((iu >> 9) << 7) | (iu & 127)
                irow[c, sl] = ((ii >> 9) << 7) | (ii & 127)

        w0 = pv[pl.ds(0, L)]
        w1 = pv[pl.ds(L, L)]
        bv = pv[pl.ds(2 * L, L)]

        for c in range(NCHUNK):
            cu = pltpu.async_copy(upk_hbm.at[urow.at[c]], ubuf, sem)
            ci = pltpu.async_copy(ipk_hbm.at[irow.at[c]], ibuf, sem)
            cu.wait()
            ci.wait()
            for g in range(CHUNK // L):
                sl = pl.ds(g * L, L)
                rows = g * L + iota
                au = ((uidx[c, sl] >> 7) & 3) << 5
                ai = ((iidx[c, sl] >> 7) & 3) << 5
                mf = jnp.zeros((L,), jnp.float32)
                d2 = jnp.zeros((L,), jnp.float32)
                for f in range(F):
                    u = plsc.load_gather(ubuf, [rows, au + f])
                    i = plsc.load_gather(ibuf, [rows, ai + f])
                    mf = mf + u * i
                    d = u - i
                    d2 = d2 + d * d
                bits = lax.bitcast_convert_type(d2, jnp.int32)
                r = lax.bitcast_convert_type(
                    jnp.int32(0x5F3759DF) - (bits >> 1), jnp.float32)
                for _ in range(3):
                    r = r * (1.5 - 0.5 * d2 * r * r)
                p2 = d2 * r
                outv[pl.ds(c * CHUNK + g * L, L)] = w0 * p2 + w1 * mf + bv

        pltpu.sync_copy(outv, out_hbm.at[pl.ds(wid * BPW, BPW)])

    return sc_kernel


_SC_KERNEL = _make_sc_kernel()


def kernel(user, item, user_emb, item_emb, W, b):
    eye = jnp.eye(128, dtype=jnp.float32)
    upk = _tc_repack(user_emb.T, eye)
    ipk = _tc_repack(item_emb.T, eye)
    user2d = user.reshape(NW * NCHUNK, CHUNK)
    item2d = item.reshape(NW * NCHUNK, CHUNK)
    params = jnp.concatenate([
        jnp.full((L,), W[0, 0], jnp.float32),
        jnp.full((L,), W[0, 1], jnp.float32),
        jnp.full((L,), b[0], jnp.float32),
    ])
    return _SC_KERNEL(user2d, item2d, upk, ipk, params)


# speedup vs baseline: 8.4149x; 1.9043x over previous
import functools

import jax
import jax.numpy as jnp
from jax import lax
from jax.experimental import pallas as pl
from jax.experimental.pallas import tpu as pltpu
from jax.experimental.pallas import tpu_sc as plsc

B = 16384
F = 32
V = 1_000_000
BV = 16384
NBLK = (V + BV - 1) // BV
VP = NBLK * (BV // 4)
NW = 32
BPW = B // NW
CHUNK = 128
NCHUNK = BPW // CHUNK
L = 16


def _tc_repack(u_t, i_t, eye):
    def one(x_ref, e, o_ref):
        for s8 in range(BV // 512):
            parts = []
            for a in range(4):
                c = x_ref[:, (s8 * 4 + a) * 128:(s8 * 4 + a + 1) * 128]
                parts.append(jax.lax.dot_general(
                    e, c, (((1,), (1,)), ((), ())),
                    preferred_element_type=jnp.float32))
            o_ref[s8 * 128:(s8 + 1) * 128, :] = jnp.concatenate(
                parts, axis=1)

    def body(u_ref, i_ref, eye_ref, ou_ref, oi_ref):
        e = eye_ref[...]
        one(u_ref, e, ou_ref)
        one(i_ref, e, oi_ref)

    return pl.pallas_call(
        body,
        grid=(NBLK,),
        in_specs=[
            pl.BlockSpec((32, BV), lambda j: (0, j)),
            pl.BlockSpec((32, BV), lambda j: (0, j)),
            pl.BlockSpec((128, 128), lambda j: (0, 0)),
        ],
        out_specs=[
            pl.BlockSpec((BV // 4, 128), lambda j: (j, 0)),
            pl.BlockSpec((BV // 4, 128), lambda j: (j, 0)),
        ],
        out_shape=[
            jax.ShapeDtypeStruct((VP, 128), jnp.float32),
            jax.ShapeDtypeStruct((VP, 128), jnp.float32),
        ],
    )(u_t, i_t, eye)


def _make_sc_kernel():
    mesh = plsc.VectorSubcoreMesh(core_axis_name="c", subcore_axis_name="s")

    @functools.partial(
        pl.kernel,
        mesh=mesh,
        out_type=jax.ShapeDtypeStruct((B,), jnp.float32),
        compiler_params=pltpu.CompilerParams(
            needs_layout_passes=False, use_tc_tiling_on_sc=True),
        scratch_types=[
            pltpu.VMEM((NCHUNK, CHUNK), jnp.int32),
            pltpu.VMEM((NCHUNK, CHUNK), jnp.int32),
            pltpu.VMEM((NCHUNK, CHUNK), jnp.int32),
            pltpu.VMEM((NCHUNK, CHUNK), jnp.int32),
            pltpu.VMEM((CHUNK, 128), jnp.float32),
            pltpu.VMEM((CHUNK, 128), jnp.float32),
            pltpu.VMEM((BPW,), jnp.float32),
            pltpu.VMEM((3 * L,), jnp.float32),
            pltpu.SemaphoreType.DMA,
        ],
    )
    def sc_kernel(user_hbm, item_hbm, upk_hbm, ipk_hbm, params_hbm,
                  out_hbm, uidx, iidx, urow, irow, ubuf, ibuf, outv, pv, sem):
        wid = lax.axis_index("s") * 2 + lax.axis_index("c")

        pltpu.sync_copy(user_hbm.at[pl.ds(wid * NCHUNK, NCHUNK)], uidx)
        pltpu.sync_copy(item_hbm.at[pl.ds(wid * NCHUNK, NCHUNK)], iidx)
        pltpu.sync_copy(params_hbm, pv)

        iota = lax.iota(jnp.int32, L)
        for c in range(NCHUNK):
            for g in range(CHUNK // L):
                sl = pl.ds(g * L, L)
                iu = uidx[c, sl]
                ii = iidx[c, sl]
                urow[c, sl] = ((iu >> 9) << 7) | (iu & 127)
                irow[c, sl] = ((ii >> 9) << 7) | (ii & 127)

        w0 = pv[pl.ds(0, L)]
        w1 = pv[pl.ds(L, L)]
        bv = pv[pl.ds(2 * L, L)]

        for c in range(NCHUNK):
            cu = pltpu.async_copy(upk_hbm.at[urow.at[c]], ubuf, sem)
            ci = pltpu.async_copy(ipk_hbm.at[irow.at[c]], ibuf, sem)
            cu.wait()
            ci.wait()
            for g in range(CHUNK // L):
                sl = pl.ds(g * L, L)
                rows = g * L + iota
                au = ((uidx[c, sl] >> 7) & 3) << 5
                ai = ((iidx[c, sl] >> 7) & 3) << 5
                mf = jnp.zeros((L,), jnp.float32)
                d2 = jnp.zeros((L,), jnp.float32)
                for f in range(F):
                    u = plsc.load_gather(ubuf, [rows, au + f])
                    i = plsc.load_gather(ibuf, [rows, ai + f])
                    mf = mf + u * i
                    d = u - i
                    d2 = d2 + d * d
                bits = lax.bitcast_convert_type(d2, jnp.int32)
                r = lax.bitcast_convert_type(
                    jnp.int32(0x5F3759DF) - (bits >> 1), jnp.float32)
                for _ in range(3):
                    r = r * (1.5 - 0.5 * d2 * r * r)
                p2 = d2 * r
                outv[pl.ds(c * CHUNK + g * L, L)] = w0 * p2 + w1 * mf + bv

        pltpu.sync_copy(outv, out_hbm.at[pl.ds(wid * BPW, BPW)])

    return sc_kernel


_SC_KERNEL = _make_sc_kernel()


def kernel(user, item, user_emb, item_emb, W, b):
    eye = jnp.eye(128, dtype=jnp.float32)
    upk, ipk = _tc_repack(user_emb.T, item_emb.T, eye)
    user2d = user.reshape(NW * NCHUNK, CHUNK)
    item2d = item.reshape(NW * NCHUNK, CHUNK)
    params = jnp.concatenate([
        jnp.full((L,), W[0, 0], jnp.float32),
        jnp.full((L,), W[0, 1], jnp.float32),
        jnp.full((L,), b[0], jnp.float32),
    ])
    return _SC_KERNEL(user2d, item2d, upk, ipk, params)


# speedup vs baseline: 8.6059x; 1.0227x over previous
import functools

import jax
import jax.numpy as jnp
from jax import lax
from jax.experimental import pallas as pl
from jax.experimental.pallas import tpu as pltpu
from jax.experimental.pallas import tpu_sc as plsc

B = 16384
F = 32
V = 1_000_000
BV = 16384
NBLK = (V + BV - 1) // BV
VP = NBLK * (BV // 4)
NW = 32
BPW = B // NW
CHUNK = 128
NCHUNK = BPW // CHUNK
L = 16


def _tc_repack(u_t, i_t, eye):
    def one(x_ref, e, o_ref):
        for s8 in range(BV // 512):
            parts = []
            for a in range(4):
                c = x_ref[:, (s8 * 4 + a) * 128:(s8 * 4 + a + 1) * 128]
                parts.append(jax.lax.dot_general(
                    e, c, (((1,), (1,)), ((), ())),
                    preferred_element_type=jnp.float32))
            o_ref[s8 * 128:(s8 + 1) * 128, :] = jnp.concatenate(
                parts, axis=1)

    def body(u_ref, i_ref, eye_ref, ou_ref, oi_ref):
        e = eye_ref[...]
        one(u_ref, e, ou_ref)
        one(i_ref, e, oi_ref)

    return pl.pallas_call(
        body,
        grid=(NBLK,),
        in_specs=[
            pl.BlockSpec((32, BV), lambda j: (0, j)),
            pl.BlockSpec((32, BV), lambda j: (0, j)),
            pl.BlockSpec((128, 128), lambda j: (0, 0)),
        ],
        out_specs=[
            pl.BlockSpec((BV // 4, 128), lambda j: (j, 0)),
            pl.BlockSpec((BV // 4, 128), lambda j: (j, 0)),
        ],
        out_shape=[
            jax.ShapeDtypeStruct((VP, 128), jnp.float32),
            jax.ShapeDtypeStruct((VP, 128), jnp.float32),
        ],
    )(u_t, i_t, eye)


def _make_sc_kernel():
    mesh = plsc.VectorSubcoreMesh(core_axis_name="c", subcore_axis_name="s")

    @functools.partial(
        pl.kernel,
        mesh=mesh,
        out_type=jax.ShapeDtypeStruct((B,), jnp.float32),
        compiler_params=pltpu.CompilerParams(
            needs_layout_passes=False, use_tc_tiling_on_sc=True),
        scratch_types=[
            pltpu.VMEM((NCHUNK, CHUNK), jnp.int32),
            pltpu.VMEM((NCHUNK, CHUNK), jnp.int32),
            pltpu.VMEM((NCHUNK, CHUNK), jnp.int32),
            pltpu.VMEM((NCHUNK, CHUNK), jnp.int32),
            pltpu.VMEM((2, CHUNK, 128), jnp.float32),
            pltpu.VMEM((2, CHUNK, 128), jnp.float32),
            pltpu.VMEM((BPW,), jnp.float32),
            pltpu.VMEM((3 * L,), jnp.float32),
            pltpu.SemaphoreType.DMA,
        ],
    )
    def sc_kernel(user_hbm, item_hbm, upk_hbm, ipk_hbm, params_hbm,
                  out_hbm, uidx, iidx, urow, irow, ubuf, ibuf, outv, pv, sem):
        wid = lax.axis_index("s") * 2 + lax.axis_index("c")

        pltpu.sync_copy(user_hbm.at[pl.ds(wid * NCHUNK, NCHUNK)], uidx)
        pltpu.sync_copy(item_hbm.at[pl.ds(wid * NCHUNK, NCHUNK)], iidx)
        pltpu.sync_copy(params_hbm, pv)

        iota = lax.iota(jnp.int32, L)
        for c in range(NCHUNK):
            for g in range(CHUNK // L):
                sl = pl.ds(g * L, L)
                iu = uidx[c, sl]
                ii = iidx[c, sl]
                urow[c, sl] = ((iu >> 9) << 7) | (iu & 127)
                irow[c, sl] = ((ii >> 9) << 7) | (ii & 127)

        w0 = pv[pl.ds(0, L)]
        w1 = pv[pl.ds(L, L)]
        bv = pv[pl.ds(2 * L, L)]

        def bf16_round(x):
            xb = lax.bitcast_convert_type(x, jnp.int32)
            xb = (xb + 0x7FFF + ((xb >> 16) & 1)) & jnp.int32(-65536)
            return lax.bitcast_convert_type(xb, jnp.float32)

        copies = [
            (pltpu.async_copy(upk_hbm.at[urow.at[0]], ubuf.at[0], sem),
             pltpu.async_copy(ipk_hbm.at[irow.at[0]], ibuf.at[0], sem)),
        ]
        for c in range(NCHUNK):
            p = c & 1
            cu, ci = copies[c]
            cu.wait()
            ci.wait()
            if c + 1 < NCHUNK:
                copies.append((
                    pltpu.async_copy(
                        upk_hbm.at[urow.at[c + 1]], ubuf.at[1 - p], sem),
                    pltpu.async_copy(
                        ipk_hbm.at[irow.at[c + 1]], ibuf.at[1 - p], sem),
                ))
            for g in range(CHUNK // L):
                sl = pl.ds(g * L, L)
                rows = g * L + iota
                au = ((uidx[c, sl] >> 7) & 3) << 5
                ai = ((iidx[c, sl] >> 7) & 3) << 5
                mf = jnp.zeros((L,), jnp.float32)
                d2 = jnp.zeros((L,), jnp.float32)
                for f in range(F):
                    u = plsc.load_gather(ubuf, [jnp.full((L,), p, jnp.int32),
                                                rows, au + f])
                    i = plsc.load_gather(ibuf, [jnp.full((L,), p, jnp.int32),
                                                rows, ai + f])
                    mf = mf + u * i
                    d = u - i
                    d2 = d2 + d * d
                bits = lax.bitcast_convert_type(d2, jnp.int32)
                r = lax.bitcast_convert_type(
                    jnp.int32(0x5F3759DF) - (bits >> 1), jnp.float32)
                for _ in range(3):
                    r = r * (1.5 - 0.5 * d2 * r * r)
                p2 = d2 * r
                outv[pl.ds(c * CHUNK + g * L, L)] = (
                    bf16_round(w0) * bf16_round(p2)
                    + bf16_round(w1) * bf16_round(mf) + bv)

        pltpu.sync_copy(outv, out_hbm.at[pl.ds(wid * BPW, BPW)])

    return sc_kernel


_SC_KERNEL = _make_sc_kernel()


def kernel(user, item, user_emb, item_emb, W, b):
    eye = jnp.eye(128, dtype=jnp.float32)
    upk, ipk = _tc_repack(user_emb.T, item_emb.T, eye)
    user2d = user.reshape(NW * NCHUNK, CHUNK)
    item2d = item.reshape(NW * NCHUNK, CHUNK)
    params = jnp.concatenate([
        jnp.full((L,), W[0, 0], jnp.float32),
        jnp.full((L,), W[0, 1], jnp.float32),
        jnp.full((L,), b[0], jnp.float32),
    ])
    return _SC_KERNEL(user2d, item2d, upk, ipk, params)


# speedup vs baseline: 9.0553x; 1.0522x over previous
import functools

import jax
import jax.numpy as jnp
from jax import lax
from jax.experimental import pallas as pl
from jax.experimental.pallas import tpu as pltpu
from jax.experimental.pallas import tpu_sc as plsc

B = 16384
F = 32
V = 1_000_000
BV = 32768
NBLK = (V + BV - 1) // BV
VP = NBLK * (BV // 4)
NW = 32
BPW = B // NW
CHUNK = 128
NCHUNK = BPW // CHUNK
L = 16


def _tc_repack(u_t, i_t, eye):
    def one(x_ref, e, o_ref):
        for s8 in range(BV // 512):
            for a in range(4):
                c = x_ref[:, (s8 * 4 + a) * 128:(s8 * 4 + a + 1) * 128]
                o_ref[s8 * 128:(s8 + 1) * 128,
                      a * 32:(a + 1) * 32] = jax.lax.dot_general(
                    e, c, (((1,), (1,)), ((), ())),
                    preferred_element_type=jnp.float32)

    def body(u_ref, i_ref, eye_ref, ou_ref, oi_ref):
        e = eye_ref[...]
        one(u_ref, e, ou_ref)
        one(i_ref, e, oi_ref)

    return pl.pallas_call(
        body,
        grid=(NBLK,),
        in_specs=[
            pl.BlockSpec((32, BV), lambda j: (0, j)),
            pl.BlockSpec((32, BV), lambda j: (0, j)),
            pl.BlockSpec((128, 128), lambda j: (0, 0)),
        ],
        out_specs=[
            pl.BlockSpec((BV // 4, 128), lambda j: (j, 0)),
            pl.BlockSpec((BV // 4, 128), lambda j: (j, 0)),
        ],
        out_shape=[
            jax.ShapeDtypeStruct((VP, 128), jnp.float32),
            jax.ShapeDtypeStruct((VP, 128), jnp.float32),
        ],
    )(u_t, i_t, eye)


def _make_sc_kernel():
    mesh = plsc.VectorSubcoreMesh(core_axis_name="c", subcore_axis_name="s")

    @functools.partial(
        pl.kernel,
        mesh=mesh,
        out_type=jax.ShapeDtypeStruct((B,), jnp.float32),
        compiler_params=pltpu.CompilerParams(
            needs_layout_passes=False, use_tc_tiling_on_sc=True),
        scratch_types=[
            pltpu.VMEM((NCHUNK, CHUNK), jnp.int32),
            pltpu.VMEM((NCHUNK, CHUNK), jnp.int32),
            pltpu.VMEM((NCHUNK, CHUNK), jnp.int32),
            pltpu.VMEM((NCHUNK, CHUNK), jnp.int32),
            pltpu.VMEM((2, CHUNK, 128), jnp.float32),
            pltpu.VMEM((2, CHUNK, 128), jnp.float32),
            pltpu.VMEM((BPW,), jnp.float32),
            pltpu.VMEM((3 * L,), jnp.float32),
            pltpu.SemaphoreType.DMA,
        ],
    )
    def sc_kernel(user_hbm, item_hbm, upk_hbm, ipk_hbm, params_hbm,
                  out_hbm, uidx, iidx, urow, irow, ubuf, ibuf, outv, pv, sem):
        wid = lax.axis_index("s") * 2 + lax.axis_index("c")

        pltpu.sync_copy(user_hbm.at[pl.ds(wid * NCHUNK, NCHUNK)], uidx)
        pltpu.sync_copy(item_hbm.at[pl.ds(wid * NCHUNK, NCHUNK)], iidx)
        pltpu.sync_copy(params_hbm, pv)

        iota = lax.iota(jnp.int32, L)
        for c in range(NCHUNK):
            for g in range(CHUNK // L):
                sl = pl.ds(g * L, L)
                iu = uidx[c, sl]
                ii = iidx[c, sl]
                urow[c, sl] = ((iu >> 9) << 7) | (iu & 127)
                irow[c, sl] = ((ii >> 9) << 7) | (ii & 127)

        w0 = pv[pl.ds(0, L)]
        w1 = pv[pl.ds(L, L)]
        bv = pv[pl.ds(2 * L, L)]

        def bf16_round(x):
            xb = lax.bitcast_convert_type(x, jnp.int32)
            xb = (xb + 0x7FFF + ((xb >> 16) & 1)) & jnp.int32(-65536)
            return lax.bitcast_convert_type(xb, jnp.float32)

        copies = [
            (pltpu.async_copy(upk_hbm.at[urow.at[0]], ubuf.at[0], sem),
             pltpu.async_copy(ipk_hbm.at[irow.at[0]], ibuf.at[0], sem)),
        ]
        for c in range(NCHUNK):
            p = c & 1
            cu, ci = copies[c]
            cu.wait()
            ci.wait()
            if c + 1 < NCHUNK:
                copies.append((
                    pltpu.async_copy(
                        upk_hbm.at[urow.at[c + 1]], ubuf.at[1 - p], sem),
                    pltpu.async_copy(
                        ipk_hbm.at[irow.at[c + 1]], ibuf.at[1 - p], sem),
                ))
            for g in range(CHUNK // L):
                sl = pl.ds(g * L, L)
                rows = g * L + iota
                au = ((uidx[c, sl] >> 7) & 3) << 5
                ai = ((iidx[c, sl] >> 7) & 3) << 5
                mf = jnp.zeros((L,), jnp.float32)
                d2 = jnp.zeros((L,), jnp.float32)
                for f in range(F):
                    u = plsc.load_gather(ubuf, [jnp.full((L,), p, jnp.int32),
                                                rows, au + f])
                    i = plsc.load_gather(ibuf, [jnp.full((L,), p, jnp.int32),
                                                rows, ai + f])
                    mf = mf + u * i
                    d = u - i
                    d2 = d2 + d * d
                bits = lax.bitcast_convert_type(d2, jnp.int32)
                r = lax.bitcast_convert_type(
                    jnp.int32(0x5F3759DF) - (bits >> 1), jnp.float32)
                for _ in range(3):
                    r = r * (1.5 - 0.5 * d2 * r * r)
                p2 = d2 * r
                outv[pl.ds(c * CHUNK + g * L, L)] = (
                    bf16_round(w0) * bf16_round(p2)
                    + bf16_round(w1) * bf16_round(mf) + bv)

        pltpu.sync_copy(outv, out_hbm.at[pl.ds(wid * BPW, BPW)])

    return sc_kernel


_SC_KERNEL = _make_sc_kernel()


def kernel(user, item, user_emb, item_emb, W, b):
    eye = jnp.eye(128, dtype=jnp.float32)
    upk, ipk = _tc_repack(user_emb.T, item_emb.T, eye)
    user2d = user.reshape(NW * NCHUNK, CHUNK)
    item2d = item.reshape(NW * NCHUNK, CHUNK)
    params = jnp.concatenate([
        jnp.full((L,), W[0, 0], jnp.float32),
        jnp.full((L,), W[0, 1], jnp.float32),
        jnp.full((L,), b[0], jnp.float32),
    ])
    return _SC_KERNEL(user2d, item2d, upk, ipk, params)
